# baseline (speedup 1.0000x reference)
import functools

import jax
import jax.numpy as jnp
from jax import lax
from jax.experimental import pallas as pl


def _mask_kernel(mask_ref, in_ref, out_ref, *, rh: int, c: int):
    ones = jnp.ones((1, c), dtype=jnp.float32)
    mf = (mask_ref[...] >= 0).astype(jnp.float32)
    for r in range(rh):
        bcast = lax.dot_general(
            mf[r:r + 1, :], ones,
            dimension_numbers=(((0,), (0,)), ((), ())),
        )
        out_ref[r] = jnp.where(bcast > 0.5, in_ref[r], 0.0)


def kernel(data_in, face_index_map):
    B, H, W, C = data_in.shape
    RH = 8
    data3 = data_in.reshape(B * H, W, C)
    mask2 = face_index_map.reshape(B * H, W)
    grid = (B * H // RH,)

    out = pl.pallas_call(
        functools.partial(_mask_kernel, rh=RH, c=C),
        grid=grid,
        in_specs=[
            pl.BlockSpec((RH, W), lambda i: (i, 0)),
            pl.BlockSpec((RH, W, C), lambda i: (i, 0, 0)),
        ],
        out_specs=pl.BlockSpec((RH, W, C), lambda i: (i, 0, 0)),
        out_shape=jax.ShapeDtypeStruct((B * H, W, C), data_in.dtype),
    )(mask2, data3)
    return out.reshape(B, H, W, C)
